# baseline (speedup 1.0000x reference)
import jax
import jax.numpy as jnp
from jax import lax
from jax.experimental import pallas as pl
from jax.experimental.pallas import tpu as pltpu
from jax.experimental.pallas import tpu_sc as plsc

N = 10000
D = 128
E = 320000
NC = 2
NS = 16
NW = NC * NS
CHUNK = 128
NBUF = 2
N_PAD = 10112
ROWS_PER_TILE = N_PAD // NS
N_CHUNKS = 80
EP_TILE = N_CHUNKS * CHUNK
E_PAD = EP_TILE * NW


def _make_sc_agg(with_deg):
  mesh = plsc.VectorSubcoreMesh(core_axis_name="c", subcore_axis_name="s")

  out_type = [jax.ShapeDtypeStruct((NC, N_PAD, D), jnp.float32)]
  if with_deg:
    out_type.append(jax.ShapeDtypeStruct((NC, N_PAD, D), jnp.float32))

  scratch = [
      pltpu.VMEM((CHUNK,), jnp.int32),
      pltpu.VMEM((CHUNK,), jnp.int32),
      pltpu.VMEM((CHUNK, D), jnp.float32),
      pltpu.VMEM_SHARED((N_PAD, D), jnp.float32),
      pltpu.SemaphoreType.DMA,
  ]
  if with_deg:
    scratch.append(pltpu.VMEM((CHUNK, D), jnp.float32))

  def body(*refs):
    if with_deg:
      (h, srcp, dstp, zrow, ones_hbm,
       agg_out, deg_out,
       src_v, dst_v, rows_v, agg_sh, sem, ones_v) = refs
    else:
      (h, srcp, dstp, zrow,
       agg_out,
       src_v, dst_v, rows_v, agg_sh, sem) = refs

    c = lax.axis_index("c")
    s = lax.axis_index("s")
    wid = c * NS + s
    my_rows = pl.ds(s * ROWS_PER_TILE, ROWS_PER_TILE)
    ebase = wid * EP_TILE

    pltpu.sync_copy(zrow, agg_sh.at[my_rows])

    if with_deg:
      pltpu.sync_copy(ones_hbm, ones_v)
      plsc.subcore_barrier()

      def dstep(j, carry):
        off = pl.multiple_of(ebase + j * CHUNK, CHUNK)
        pltpu.sync_copy(dstp.at[pl.ds(off, CHUNK)], dst_v)
        pltpu.sync_copy(ones_v, agg_sh.at[dst_v], add=True)
        return carry

      lax.fori_loop(0, N_CHUNKS, dstep, 0)
      plsc.subcore_barrier()
      pltpu.sync_copy(agg_sh.at[my_rows], deg_out.at[c].at[my_rows])
      plsc.subcore_barrier()
      pltpu.sync_copy(zrow, agg_sh.at[my_rows])

    plsc.subcore_barrier()

    def step(j, carry):
      off = pl.multiple_of(ebase + j * CHUNK, CHUNK)
      pltpu.sync_copy(srcp.at[pl.ds(off, CHUNK)], src_v)
      pltpu.sync_copy(dstp.at[pl.ds(off, CHUNK)], dst_v)
      pltpu.async_copy(h.at[src_v], rows_v, sem).wait()
      pltpu.sync_copy(rows_v, agg_sh.at[dst_v], add=True)
      return carry

    lax.fori_loop(0, N_CHUNKS, step, 0)
    plsc.subcore_barrier()

    pltpu.sync_copy(agg_sh.at[my_rows], agg_out.at[c].at[my_rows])

  return pl.kernel(body, out_type=tuple(out_type) if with_deg else out_type[0],
                   mesh=mesh, scratch_types=scratch)


_sc_agg_deg = _make_sc_agg(True)
_sc_agg = _make_sc_agg(False)


BM = 400


def _tc_self(x, Ws, b):
  def body(x_r, ws_r, b_r, o_r):
    o_r[...] = jnp.dot(x_r[...], ws_r[...],
                       preferred_element_type=jnp.float32) + b_r[...]

  return pl.pallas_call(
      body,
      grid=(N // BM,),
      in_specs=[
          pl.BlockSpec((BM, D), lambda i: (i, 0)),
          pl.BlockSpec((D, D), lambda i: (0, 0)),
          pl.BlockSpec((1, D), lambda i: (0, 0)),
      ],
      out_specs=pl.BlockSpec((BM, D), lambda i: (i, 0)),
      out_shape=jax.ShapeDtypeStruct((N, D), jnp.float32),
  )(x, Ws, b.reshape(1, D))


def _tc_combine(yself, p0, p1, d0, d1, Wn, relu):
  def body(y_r, p0_r, p1_r, d0_r, d1_r, wn_r, o_r):
    deg = jnp.maximum(d0_r[:, 0:1] + d1_r[:, 0:1], 1.0)
    agg = (p0_r[...] + p1_r[...]) / deg
    y = y_r[...] + jnp.dot(agg, wn_r[...], preferred_element_type=jnp.float32)
    if relu:
      y = jnp.maximum(y, 0.0)
    o_r[...] = y

  return pl.pallas_call(
      body,
      grid=(N // BM,),
      in_specs=[
          pl.BlockSpec((BM, D), lambda i: (i, 0)),
          pl.BlockSpec((BM, D), lambda i: (i, 0)),
          pl.BlockSpec((BM, D), lambda i: (i, 0)),
          pl.BlockSpec((BM, D), lambda i: (i, 0)),
          pl.BlockSpec((BM, D), lambda i: (i, 0)),
          pl.BlockSpec((D, D), lambda i: (0, 0)),
      ],
      out_specs=pl.BlockSpec((BM, D), lambda i: (i, 0)),
      out_shape=jax.ShapeDtypeStruct((N, D), jnp.float32),
  )(yself, p0, p1, d0, d1, Wn)


def kernel(x, edge_index, W_self1, W_neigh1, b1, W_self2, W_neigh2, b2):
  src = edge_index[0].astype(jnp.int32)
  dst = edge_index[1].astype(jnp.int32)
  pad = E_PAD - E
  src_p = jnp.concatenate([src, jnp.zeros((pad,), jnp.int32)])
  dst_p = jnp.concatenate([dst, jnp.full((pad,), N, jnp.int32)])

  zrow = jnp.zeros((ROWS_PER_TILE, D), jnp.float32)
  ones_h = jnp.ones((CHUNK, D), jnp.float32)

  agg1, degp = _sc_agg_deg(x, src_p, dst_p, zrow, ones_h)
  self1 = _tc_self(x, W_self1, b1)
  d0, d1 = degp[0, :N], degp[1, :N]
  h1 = _tc_combine(self1, agg1[0, :N], agg1[1, :N], d0, d1,
                   W_neigh1, relu=True)

  agg2 = _sc_agg(h1, src_p, dst_p, zrow)
  self2 = _tc_self(h1, W_self2, b2)
  out = _tc_combine(self2, agg2[0, :N], agg2[1, :N], d0, d1,
                    W_neigh2, relu=False)
  return out


# speedup vs baseline: 1.3629x; 1.3629x over previous
import jax
import jax.numpy as jnp
from jax import lax
from jax.experimental import pallas as pl
from jax.experimental.pallas import tpu as pltpu
from jax.experimental.pallas import tpu_sc as plsc

N = 10000
D = 128
E = 320000
NC = 2
NS = 16
NW = NC * NS
CHUNK = 128
N_PAD = 10112
ROWS_PER_TILE = N_PAD // NS
E_PAD = ((E + NW * CHUNK - 1) // (NW * CHUNK)) * (NW * CHUNK)
EP_TILE = E_PAD // NW
N_CHUNKS = EP_TILE // CHUNK


def _make_sc_agg(with_deg):
  mesh = plsc.VectorSubcoreMesh(core_axis_name="c", subcore_axis_name="s")

  out_type = [jax.ShapeDtypeStruct((NC, N_PAD, D), jnp.float32)]
  if with_deg:
    out_type.append(jax.ShapeDtypeStruct((NC, N_PAD, D), jnp.float32))

  scratch = [
      pltpu.VMEM((CHUNK,), jnp.int32),
      pltpu.VMEM((CHUNK,), jnp.int32),
      pltpu.VMEM((CHUNK, D), jnp.float32),
      pltpu.VMEM_SHARED((N_PAD, D), jnp.float32),
      pltpu.SemaphoreType.DMA,
  ]
  if with_deg:
    scratch.append(pltpu.VMEM((CHUNK, D), jnp.float32))

  def body(*refs):
    if with_deg:
      (h, srcp, dstp, zrow, ones_hbm,
       agg_out, deg_out,
       src_v, dst_v, rows_v, agg_sh, sem, ones_v) = refs
    else:
      (h, srcp, dstp, zrow,
       agg_out,
       src_v, dst_v, rows_v, agg_sh, sem) = refs

    c = lax.axis_index("c")
    s = lax.axis_index("s")
    wid = c * NS + s
    my_rows = pl.ds(s * ROWS_PER_TILE, ROWS_PER_TILE)
    ebase = wid * EP_TILE

    pltpu.sync_copy(zrow, agg_sh.at[my_rows])

    if with_deg:
      pltpu.sync_copy(ones_hbm, ones_v)
      plsc.subcore_barrier()

      def dstep(j, carry):
        off = pl.multiple_of(ebase + j * CHUNK, CHUNK)
        pltpu.sync_copy(dstp.at[pl.ds(off, CHUNK)], dst_v)
        pltpu.sync_copy(ones_v, agg_sh.at[dst_v], add=True)
        return carry

      lax.fori_loop(0, N_CHUNKS, dstep, 0)
      plsc.subcore_barrier()
      pltpu.sync_copy(agg_sh.at[my_rows], deg_out.at[c].at[my_rows])
      plsc.subcore_barrier()
      pltpu.sync_copy(zrow, agg_sh.at[my_rows])

    plsc.subcore_barrier()

    def step(j, carry):
      off = pl.multiple_of(ebase + j * CHUNK, CHUNK)
      pltpu.sync_copy(srcp.at[pl.ds(off, CHUNK)], src_v)
      pltpu.sync_copy(dstp.at[pl.ds(off, CHUNK)], dst_v)
      pltpu.async_copy(h.at[src_v], rows_v, sem).wait()
      pltpu.sync_copy(rows_v, agg_sh.at[dst_v], add=True)
      return carry

    lax.fori_loop(0, N_CHUNKS, step, 0)
    plsc.subcore_barrier()

    pltpu.sync_copy(agg_sh.at[my_rows], agg_out.at[c].at[my_rows])

  return pl.kernel(body, out_type=tuple(out_type) if with_deg else out_type[0],
                   mesh=mesh, scratch_types=scratch)


_sc_agg_deg = _make_sc_agg(True)
_sc_agg = _make_sc_agg(False)


def _tc_layer(x, p0, p1, d0, d1, Ws, Wn, b, relu):
  BM = 400

  def body(x_r, p0_r, p1_r, d0_r, d1_r, ws_r, wn_r, b_r, o_r):
    deg = jnp.maximum(d0_r[:, 0:1] + d1_r[:, 0:1], 1.0)
    agg = (p0_r[...] + p1_r[...]) / deg
    y = (jnp.dot(x_r[...], ws_r[...], preferred_element_type=jnp.float32)
         + jnp.dot(agg, wn_r[...], preferred_element_type=jnp.float32)
         + b_r[...])
    if relu:
      y = jnp.maximum(y, 0.0)
    o_r[...] = y

  return pl.pallas_call(
      body,
      grid=(N // BM,),
      in_specs=[
          pl.BlockSpec((BM, D), lambda i: (i, 0)),
          pl.BlockSpec((BM, D), lambda i: (i, 0)),
          pl.BlockSpec((BM, D), lambda i: (i, 0)),
          pl.BlockSpec((BM, D), lambda i: (i, 0)),
          pl.BlockSpec((BM, D), lambda i: (i, 0)),
          pl.BlockSpec((D, D), lambda i: (0, 0)),
          pl.BlockSpec((D, D), lambda i: (0, 0)),
          pl.BlockSpec((1, D), lambda i: (0, 0)),
      ],
      out_specs=pl.BlockSpec((BM, D), lambda i: (i, 0)),
      out_shape=jax.ShapeDtypeStruct((N, D), jnp.float32),
  )(x, p0, p1, d0, d1, Ws, Wn, b.reshape(1, D))


def kernel(x, edge_index, W_self1, W_neigh1, b1, W_self2, W_neigh2, b2):
  src = edge_index[0].astype(jnp.int32)
  dst = edge_index[1].astype(jnp.int32)
  pad = E_PAD - E
  src_p = jnp.concatenate([src, jnp.zeros((pad,), jnp.int32)])
  dst_p = jnp.concatenate([dst, jnp.full((pad,), N, jnp.int32)])

  zrow = jnp.zeros((ROWS_PER_TILE, D), jnp.float32)
  ones_h = jnp.ones((CHUNK, D), jnp.float32)

  agg1, degp = _sc_agg_deg(x, src_p, dst_p, zrow, ones_h)
  d0, d1 = degp[0, :N], degp[1, :N]
  h1 = _tc_layer(x, agg1[0, :N], agg1[1, :N], d0, d1,
                 W_self1, W_neigh1, b1, relu=True)

  agg2 = _sc_agg(h1, src_p, dst_p, zrow)
  out = _tc_layer(h1, agg2[0, :N], agg2[1, :N], d0, d1,
                  W_self2, W_neigh2, b2, relu=False)
  return out


# speedup vs baseline: 1.3903x; 1.0201x over previous
import jax
import jax.numpy as jnp
from jax import lax
from jax.experimental import pallas as pl
from jax.experimental.pallas import tpu as pltpu
from jax.experimental.pallas import tpu_sc as plsc

N = 10000
D = 128
E = 320000
NC = 2
NS = 16
NW = NC * NS
CHUNK = 128
N_PAD = 10112
ROWS_PER_TILE = N_PAD // NS
E_PAD = ((E + NW * CHUNK - 1) // (NW * CHUNK)) * (NW * CHUNK)
EP_TILE = E_PAD // NW
N_CHUNKS = EP_TILE // CHUNK


def _make_sc_agg(with_deg):
  mesh = plsc.VectorSubcoreMesh(core_axis_name="c", subcore_axis_name="s")

  out_type = [jax.ShapeDtypeStruct((NC, N_PAD, D), jnp.float32)]
  if with_deg:
    out_type.append(jax.ShapeDtypeStruct((NC, N_PAD, D), jnp.float32))

  scratch = [
      pltpu.VMEM((CHUNK,), jnp.int32),
      pltpu.VMEM((CHUNK,), jnp.int32),
      pltpu.VMEM((CHUNK, D), jnp.float32),
      pltpu.VMEM_SHARED((N_PAD, D), jnp.float32),
      pltpu.SemaphoreType.DMA,
  ]
  if with_deg:
    scratch.append(pltpu.VMEM((CHUNK, D), jnp.float32))

  def body(*refs):
    if with_deg:
      (h, srcp, dstp, zrow, ones_hbm,
       agg_out, deg_out,
       src_v, dst_v, rows_v, agg_sh, sem, ones_v) = refs
    else:
      (h, srcp, dstp, zrow,
       agg_out,
       src_v, dst_v, rows_v, agg_sh, sem) = refs

    c = lax.axis_index("c")
    s = lax.axis_index("s")
    wid = c * NS + s
    my_rows = pl.ds(s * ROWS_PER_TILE, ROWS_PER_TILE)
    ebase = wid * EP_TILE

    pltpu.sync_copy(zrow, agg_sh.at[my_rows])

    if with_deg:
      pltpu.sync_copy(ones_hbm, ones_v)
      plsc.subcore_barrier()

      def dstep(j, carry):
        off = pl.multiple_of(ebase + j * CHUNK, CHUNK)
        pltpu.sync_copy(dstp.at[pl.ds(off, CHUNK)], dst_v)
        pltpu.sync_copy(ones_v, agg_sh.at[dst_v], add=True)
        return carry

      lax.fori_loop(0, N_CHUNKS, dstep, 0)
      plsc.subcore_barrier()
      pltpu.sync_copy(agg_sh.at[my_rows], deg_out.at[c].at[my_rows])
      plsc.subcore_barrier()
      pltpu.sync_copy(zrow, agg_sh.at[my_rows])

    plsc.subcore_barrier()

    def step(j, carry):
      off = pl.multiple_of(ebase + j * CHUNK, CHUNK)
      pltpu.sync_copy(srcp.at[pl.ds(off, CHUNK)], src_v)
      pltpu.sync_copy(dstp.at[pl.ds(off, CHUNK)], dst_v)
      pltpu.async_copy(h.at[src_v], rows_v, sem).wait()
      pltpu.sync_copy(rows_v, agg_sh.at[dst_v], add=True)
      return carry

    lax.fori_loop(0, N_CHUNKS, step, 0)
    plsc.subcore_barrier()

    pltpu.sync_copy(agg_sh.at[my_rows], agg_out.at[c].at[my_rows])

  return pl.kernel(body, out_type=tuple(out_type) if with_deg else out_type[0],
                   mesh=mesh, scratch_types=scratch)


_sc_agg_deg = _make_sc_agg(True)
_sc_agg = _make_sc_agg(False)


def _tc_layer(x, p0, p1, d0, d1, Ws, Wn, b, relu):
  BM = 2000

  def body(x_r, p0_r, p1_r, d0_r, d1_r, ws_r, wn_r, b_r, o_r):
    deg = jnp.maximum(d0_r[:, 0:1] + d1_r[:, 0:1], 1.0)
    agg = (p0_r[...] + p1_r[...]) / deg
    y = (jnp.dot(x_r[...], ws_r[...], preferred_element_type=jnp.float32)
         + jnp.dot(agg, wn_r[...], preferred_element_type=jnp.float32)
         + b_r[...])
    if relu:
      y = jnp.maximum(y, 0.0)
    o_r[...] = y

  return pl.pallas_call(
      body,
      grid=(N // BM,),
      in_specs=[
          pl.BlockSpec((BM, D), lambda i: (i, 0)),
          pl.BlockSpec((BM, D), lambda i: (i, 0)),
          pl.BlockSpec((BM, D), lambda i: (i, 0)),
          pl.BlockSpec((BM, D), lambda i: (i, 0)),
          pl.BlockSpec((BM, D), lambda i: (i, 0)),
          pl.BlockSpec((D, D), lambda i: (0, 0)),
          pl.BlockSpec((D, D), lambda i: (0, 0)),
          pl.BlockSpec((1, D), lambda i: (0, 0)),
      ],
      out_specs=pl.BlockSpec((BM, D), lambda i: (i, 0)),
      out_shape=jax.ShapeDtypeStruct((N, D), jnp.float32),
  )(x, p0, p1, d0, d1, Ws, Wn, b.reshape(1, D))


def kernel(x, edge_index, W_self1, W_neigh1, b1, W_self2, W_neigh2, b2):
  src = edge_index[0].astype(jnp.int32)
  dst = edge_index[1].astype(jnp.int32)
  pad = E_PAD - E
  src_p = jnp.concatenate([src, jnp.zeros((pad,), jnp.int32)])
  dst_p = jnp.concatenate([dst, jnp.full((pad,), N, jnp.int32)])

  zrow = jnp.zeros((ROWS_PER_TILE, D), jnp.float32)
  ones_h = jnp.ones((CHUNK, D), jnp.float32)

  agg1, degp = _sc_agg_deg(x, src_p, dst_p, zrow, ones_h)
  d0, d1 = degp[0, :N], degp[1, :N]
  h1 = _tc_layer(x, agg1[0, :N], agg1[1, :N], d0, d1,
                 W_self1, W_neigh1, b1, relu=True)

  agg2 = _sc_agg(h1, src_p, dst_p, zrow)
  out = _tc_layer(h1, agg2[0, :N], agg2[1, :N], d0, d1,
                  W_self2, W_neigh2, b2, relu=False)
  return out


# speedup vs baseline: 1.3928x; 1.0018x over previous
import jax
import jax.numpy as jnp
from jax import lax
from jax.experimental import pallas as pl
from jax.experimental.pallas import tpu as pltpu
from jax.experimental.pallas import tpu_sc as plsc

N = 10000
D = 128
E = 320000
NC = 2
NS = 16
NW = NC * NS
CHUNK = 128
N_PAD = 10112
ROWS_PER_TILE = N_PAD // NS
E_PAD = ((E + NW * CHUNK - 1) // (NW * CHUNK)) * (NW * CHUNK)
EP_TILE = E_PAD // NW
N_CHUNKS = EP_TILE // CHUNK


def _make_sc_agg(with_deg):
  mesh = plsc.VectorSubcoreMesh(core_axis_name="c", subcore_axis_name="s")

  out_type = [jax.ShapeDtypeStruct((NC, N_PAD, D), jnp.float32)]
  if with_deg:
    out_type.append(jax.ShapeDtypeStruct((NC, N_PAD, D), jnp.float32))

  scratch = [
      pltpu.VMEM((CHUNK,), jnp.int32),
      pltpu.VMEM((CHUNK,), jnp.int32),
      pltpu.VMEM((CHUNK, D), jnp.float32),
      pltpu.VMEM_SHARED((N_PAD, D), jnp.float32),
      pltpu.SemaphoreType.DMA,
  ]
  if with_deg:
    scratch.append(pltpu.VMEM((CHUNK, D), jnp.float32))

  def body(*refs):
    if with_deg:
      (h, srcp, dstp, zrow, ones_hbm,
       agg_out, deg_out,
       src_v, dst_v, rows_v, agg_sh, sem, ones_v) = refs
    else:
      (h, srcp, dstp, zrow,
       agg_out,
       src_v, dst_v, rows_v, agg_sh, sem) = refs

    c = lax.axis_index("c")
    s = lax.axis_index("s")
    wid = c * NS + s
    my_rows = pl.ds(s * ROWS_PER_TILE, ROWS_PER_TILE)
    ebase = wid * EP_TILE

    pltpu.sync_copy(zrow, agg_sh.at[my_rows])

    if with_deg:
      pltpu.sync_copy(ones_hbm, ones_v)
      plsc.subcore_barrier()

      def dstep(j, carry):
        off = pl.multiple_of(ebase + j * CHUNK, CHUNK)
        pltpu.sync_copy(dstp.at[pl.ds(off, CHUNK)], dst_v)
        pltpu.sync_copy(ones_v, agg_sh.at[dst_v], add=True)
        return carry

      lax.fori_loop(0, N_CHUNKS, dstep, 0)
      plsc.subcore_barrier()
      pltpu.sync_copy(agg_sh.at[my_rows], deg_out.at[c].at[my_rows])
      plsc.subcore_barrier()
      pltpu.sync_copy(zrow, agg_sh.at[my_rows])

    plsc.subcore_barrier()

    def step(j, carry):
      off = pl.multiple_of(ebase + j * CHUNK, CHUNK)
      pltpu.sync_copy(srcp.at[pl.ds(off, CHUNK)], src_v)
      pltpu.sync_copy(dstp.at[pl.ds(off, CHUNK)], dst_v)
      pltpu.async_copy(h.at[src_v], rows_v, sem).wait()
      pltpu.sync_copy(rows_v, agg_sh.at[dst_v], add=True)
      return carry

    lax.fori_loop(0, N_CHUNKS, step, 0)
    plsc.subcore_barrier()

    pltpu.sync_copy(agg_sh.at[my_rows], agg_out.at[c].at[my_rows])

  return pl.kernel(body, out_type=tuple(out_type) if with_deg else out_type[0],
                   mesh=mesh, scratch_types=scratch)


_sc_agg_deg = _make_sc_agg(True)
_sc_agg = _make_sc_agg(False)


def _tc_layer(x, p0, p1, d0, d1, Ws, Wn, b, relu):
  BM = 5000

  def body(x_r, p0_r, p1_r, d0_r, d1_r, ws_r, wn_r, b_r, o_r):
    deg = jnp.maximum(d0_r[:, 0:1] + d1_r[:, 0:1], 1.0)
    agg = (p0_r[...] + p1_r[...]) / deg
    y = (jnp.dot(x_r[...], ws_r[...], preferred_element_type=jnp.float32)
         + jnp.dot(agg, wn_r[...], preferred_element_type=jnp.float32)
         + b_r[...])
    if relu:
      y = jnp.maximum(y, 0.0)
    o_r[...] = y

  return pl.pallas_call(
      body,
      grid=(N // BM,),
      in_specs=[
          pl.BlockSpec((BM, D), lambda i: (i, 0)),
          pl.BlockSpec((BM, D), lambda i: (i, 0)),
          pl.BlockSpec((BM, D), lambda i: (i, 0)),
          pl.BlockSpec((BM, D), lambda i: (i, 0)),
          pl.BlockSpec((BM, D), lambda i: (i, 0)),
          pl.BlockSpec((D, D), lambda i: (0, 0)),
          pl.BlockSpec((D, D), lambda i: (0, 0)),
          pl.BlockSpec((1, D), lambda i: (0, 0)),
      ],
      out_specs=pl.BlockSpec((BM, D), lambda i: (i, 0)),
      out_shape=jax.ShapeDtypeStruct((N, D), jnp.float32),
  )(x, p0, p1, d0, d1, Ws, Wn, b.reshape(1, D))


def kernel(x, edge_index, W_self1, W_neigh1, b1, W_self2, W_neigh2, b2):
  src = edge_index[0].astype(jnp.int32)
  dst = edge_index[1].astype(jnp.int32)
  pad = E_PAD - E
  src_p = jnp.concatenate([src, jnp.zeros((pad,), jnp.int32)])
  dst_p = jnp.concatenate([dst, jnp.full((pad,), N, jnp.int32)])

  zrow = jnp.zeros((ROWS_PER_TILE, D), jnp.float32)
  ones_h = jnp.ones((CHUNK, D), jnp.float32)

  agg1, degp = _sc_agg_deg(x, src_p, dst_p, zrow, ones_h)
  d0, d1 = degp[0, :N], degp[1, :N]
  h1 = _tc_layer(x, agg1[0, :N], agg1[1, :N], d0, d1,
                 W_self1, W_neigh1, b1, relu=True)

  agg2 = _sc_agg(h1, src_p, dst_p, zrow)
  out = _tc_layer(h1, agg2[0, :N], agg2[1, :N], d0, d1,
                  W_self2, W_neigh2, b2, relu=False)
  return out
